# Initial kernel scaffold; baseline (speedup 1.0000x reference)
import functools

import jax
import jax.numpy as jnp
from jax import lax
from jax.experimental import pallas as pl
from jax.experimental.pallas import tpu as pltpu
from jax.experimental.pallas import tpu_sc as plsc

N = 10000
E = 320000
D_IN = 128
HID = 16

NC = 2
NS = 16
NW = NC * NS
LANES = 16

EB = 128
ETOT = E + N
EPW = 10368
NBLK = EPW // EB
ETOT_PAD = EPW * NW
NPAD = 10016
ROWS_PER_TILE = NPAD // NS
OUT_PER_TILE = N // NS

_GDN = lax.GatherDimensionNumbers(
    offset_dims=(), collapsed_slice_dims=(0,), start_index_map=(0,))


def _shuf(v, perm):
  return lax.gather(v, perm.reshape(LANES, 1), dimension_numbers=_GDN,
                    slice_sizes=(1,),
                    mode=lax.GatherScatterMode.PROMISE_IN_BOUNDS)


def _edge_kernel_body(xors, srcp, dstp, xl, xr, atth, den_out, msg_out,
                      attb, sidx, didx, xlb, xrb, exb, msgb, zbuf,
                      den_sh, msg_sh, sem1, sem2):
  c = lax.axis_index("c")
  s = lax.axis_index("s")
  wid = s * NC + c

  zero = jnp.zeros((LANES,), jnp.float32)

  def zb(i, carry):
    zbuf[i, :] = zero
    return carry

  lax.fori_loop(0, ROWS_PER_TILE // 2, zb, 0)
  zbase = s * ROWS_PER_TILE
  half = ROWS_PER_TILE // 2
  pltpu.sync_copy(zbuf, den_sh.at[pl.ds(zbase, half)])
  pltpu.sync_copy(zbuf, den_sh.at[pl.ds(zbase + half, half)])
  pltpu.sync_copy(zbuf, msg_sh.at[pl.ds(zbase, half)])
  pltpu.sync_copy(zbuf, msg_sh.at[pl.ds(zbase + half, half)])
  pltpu.sync_copy(atth, attb)
  plsc.subcore_barrier()

  att = attb[...]
  perms = [jnp.arange(LANES, dtype=jnp.int32) ^ x for x in xors]

  def blk(b, carry):
    base = wid * EPW + b * EB
    pltpu.sync_copy(srcp.at[pl.ds(base, EB)], sidx)
    pltpu.sync_copy(dstp.at[pl.ds(base, EB)], didx)
    pltpu.async_copy(xl.at[sidx], xlb, sem1).wait()
    pltpu.async_copy(xr.at[didx], xrb, sem2).wait()

    def edge(i, ecarry):
      vl = xlb[i, :]
      sv = vl + xrb[i, :]
      lk = jnp.where(sv > 0, sv, sv * jnp.float32(0.2))
      p = lk * att
      for perm in perms:
        p = p + _shuf(p, perm)
      ex = jnp.exp(p)
      exb[i, :] = ex
      msgb[i, :] = ex * vl
      return ecarry

    lax.fori_loop(0, EB, edge, 0)
    pltpu.sync_copy(exb, den_sh.at[didx], add=True)
    pltpu.sync_copy(msgb, msg_sh.at[didx], add=True)
    return carry

  lax.fori_loop(0, NBLK, blk, 0)
  plsc.subcore_barrier()

  ob = s * OUT_PER_TILE
  pltpu.sync_copy(den_sh.at[pl.ds(ob, OUT_PER_TILE)],
                  den_out.at[c, pl.ds(ob, OUT_PER_TILE)])
  pltpu.sync_copy(msg_sh.at[pl.ds(ob, OUT_PER_TILE)],
                  msg_out.at[c, pl.ds(ob, OUT_PER_TILE)])


def _make_edge_kernel(xors):
  mesh = plsc.VectorSubcoreMesh(core_axis_name="c", subcore_axis_name="s",
                                num_cores=NC, num_subcores=NS)
  return pl.kernel(
      functools.partial(_edge_kernel_body, xors),
      out_type=[
          jax.ShapeDtypeStruct((NC, N, HID), jnp.float32),
          jax.ShapeDtypeStruct((NC, N, HID), jnp.float32),
      ],
      mesh=mesh,
      scratch_types=[
          pltpu.VMEM((LANES,), jnp.float32),
          pltpu.VMEM((EB,), jnp.int32),
          pltpu.VMEM((EB,), jnp.int32),
          pltpu.VMEM((EB, HID), jnp.float32),
          pltpu.VMEM((EB, HID), jnp.float32),
          pltpu.VMEM((EB, HID), jnp.float32),
          pltpu.VMEM((EB, HID), jnp.float32),
          pltpu.VMEM((ROWS_PER_TILE // 2, HID), jnp.float32),
          pltpu.VMEM_SHARED((NPAD, HID), jnp.float32),
          pltpu.VMEM_SHARED((NPAD, HID), jnp.float32),
          pltpu.SemaphoreType.DMA,
          pltpu.SemaphoreType.DMA,
      ],
      name="gat_edge_pass",
  )


_edge_l1 = _make_edge_kernel((1, 2))
_edge_l2 = _make_edge_kernel((1, 2, 4, 8))


def _proj1_body(x_ref, w_ref, b_ref, ol_ref, or_ref):
  acc = jnp.dot(x_ref[...], w_ref[...],
                preferred_element_type=jnp.float32) + b_ref[...]
  ol_ref[...] = acc[:, :HID]
  or_ref[...] = acc[:, HID:]


_PROJ_ROWS = 512
_NROWS1 = 10240


def _proj1(xpad, wcat, bcat):
  return pl.pallas_call(
      _proj1_body,
      grid=(_NROWS1 // _PROJ_ROWS,),
      in_specs=[
          pl.BlockSpec((_PROJ_ROWS, D_IN), lambda i: (i, 0)),
          pl.BlockSpec((D_IN, 2 * HID), lambda i: (0, 0)),
          pl.BlockSpec((1, 2 * HID), lambda i: (0, 0)),
      ],
      out_specs=[
          pl.BlockSpec((_PROJ_ROWS, HID), lambda i: (i, 0)),
          pl.BlockSpec((_PROJ_ROWS, HID), lambda i: (i, 0)),
      ],
      out_shape=[
          jax.ShapeDtypeStruct((_NROWS1, HID), jnp.float32),
          jax.ShapeDtypeStruct((_NROWS1, HID), jnp.float32),
      ],
  )(xpad, wcat, bcat)


_FUSE_ROWS = 500


def _fuse_body(den_ref, msg_ref, b1_ref, w_ref, b2_ref, ol_ref, or_ref):
  den = den_ref[0] + den_ref[1]
  msg = msg_ref[0] + msg_ref[1]
  h = jnp.maximum(msg / (den + 1e-16) + b1_ref[...], 0.0)
  acc = jnp.dot(h, w_ref[...], preferred_element_type=jnp.float32) + b2_ref[...]
  ol_ref[...] = acc[:, :HID]
  or_ref[...] = acc[:, HID:]


def _fuse(den, msg, bias1, wcat2, bcat2):
  return pl.pallas_call(
      _fuse_body,
      grid=(N // _FUSE_ROWS,),
      in_specs=[
          pl.BlockSpec((NC, _FUSE_ROWS, HID), lambda i: (0, i, 0)),
          pl.BlockSpec((NC, _FUSE_ROWS, HID), lambda i: (0, i, 0)),
          pl.BlockSpec((1, HID), lambda i: (0, 0)),
          pl.BlockSpec((HID, 2 * HID), lambda i: (0, 0)),
          pl.BlockSpec((1, 2 * HID), lambda i: (0, 0)),
      ],
      out_specs=[
          pl.BlockSpec((_FUSE_ROWS, HID), lambda i: (i, 0)),
          pl.BlockSpec((_FUSE_ROWS, HID), lambda i: (i, 0)),
      ],
      out_shape=[
          jax.ShapeDtypeStruct((N, HID), jnp.float32),
          jax.ShapeDtypeStruct((N, HID), jnp.float32),
      ],
  )(den, msg, bias1, wcat2, bcat2)


def _final_body(den_ref, msg_ref, b_ref, o_ref):
  den = den_ref[0] + den_ref[1]
  msg = msg_ref[0] + msg_ref[1]
  o_ref[...] = msg / (den + 1e-16) + b_ref[...]


def _final(den, msg, bias2):
  return pl.pallas_call(
      _final_body,
      grid=(N // _FUSE_ROWS,),
      in_specs=[
          pl.BlockSpec((NC, _FUSE_ROWS, HID), lambda i: (0, i, 0)),
          pl.BlockSpec((NC, _FUSE_ROWS, HID), lambda i: (0, i, 0)),
          pl.BlockSpec((1, HID), lambda i: (0, 0)),
      ],
      out_specs=pl.BlockSpec((_FUSE_ROWS, HID), lambda i: (i, 0)),
      out_shape=jax.ShapeDtypeStruct((N, HID), jnp.float32),
  )(den, msg, bias2)


@jax.jit
def _impl(x, edge_index, Wl1, bl1, Wr1, br1, att1, bias1,
          Wl2, bl2, Wr2, br2, att2, bias2):
  loop = jnp.arange(N, dtype=edge_index.dtype)
  pad = jnp.full((ETOT_PAD - ETOT,), N, dtype=edge_index.dtype)
  srcp = jnp.concatenate([edge_index[0], loop, pad])
  dstp = jnp.concatenate([edge_index[1], loop, pad])

  xpad = jnp.pad(x, ((0, _NROWS1 - N), (0, 0)))
  w1 = jnp.concatenate([Wl1, Wr1], axis=1)
  b1 = jnp.concatenate([bl1, br1]).reshape(1, 2 * HID)
  xl1, xr1 = _proj1(xpad, w1, b1)
  xl1 = xl1[:NPAD]
  xr1 = xr1[:NPAD]

  att1v = att1.reshape(HID)
  den1, msg1 = _edge_l1(srcp, dstp, xl1, xr1, att1v)

  w2 = jnp.concatenate([Wl2, Wr2], axis=1)
  b2 = jnp.concatenate([bl2, br2]).reshape(1, 2 * HID)
  xl2, xr2 = _fuse(den1, msg1, bias1.reshape(1, HID), w2, b2)
  xl2 = jnp.pad(xl2, ((0, NPAD - N), (0, 0)))
  xr2 = jnp.pad(xr2, ((0, NPAD - N), (0, 0)))

  att2v = att2.reshape(HID)
  den2, msg2 = _edge_l2(srcp, dstp, xl2, xr2, att2v)

  return _final(den2, msg2, bias2.reshape(1, HID))


def kernel(x, edge_index, Wl1, bl1, Wr1, br1, att1, bias1,
           Wl2, bl2, Wr2, br2, att2, bias2):
  return _impl(x, edge_index, Wl1, bl1, Wr1, br1, att1, bias1,
               Wl2, bl2, Wr2, br2, att2, bias2)


# speedup vs baseline: 73.7761x; 73.7761x over previous
import functools

import jax
import jax.numpy as jnp
from jax import lax
from jax.experimental import pallas as pl
from jax.experimental.pallas import tpu as pltpu
from jax.experimental.pallas import tpu_sc as plsc

N = 10000
E = 320000
D_IN = 128
HID = 16

NC = 2
NS = 16
NW = NC * NS
LANES = 16

EB = 128
ETOT = E + N
EPW = 10368
NBLK = EPW // EB
ETOT_PAD = EPW * NW
NPAD = 10016
ACC_ROWS = 10240
ROWS_PER_TILE = ACC_ROWS // NS

_GDN = lax.GatherDimensionNumbers(
    offset_dims=(), collapsed_slice_dims=(0,), start_index_map=(0,))


def _shuf(v, perm):
  return lax.gather(v, perm.reshape(LANES, 1), dimension_numbers=_GDN,
                    slice_sizes=(1,),
                    mode=lax.GatherScatterMode.PROMISE_IN_BOUNDS)


def _edge_kernel_body(xors, srcp, dstp, xl, xr, atth, den_out, msg_out,
                      attb, sidx, didx, xlb, xrb, exb, msgb, zbuf,
                      den_sh, msg_sh, sem1, sem2):
  c = lax.axis_index("c")
  s = lax.axis_index("s")
  wid = s * NC + c

  zero = jnp.zeros((LANES,), jnp.float32)

  def zb(i, carry):
    zbuf[i, :] = zero
    return carry

  lax.fori_loop(0, ROWS_PER_TILE // 2, zb, 0)
  zbase = s * ROWS_PER_TILE
  half = ROWS_PER_TILE // 2
  pltpu.sync_copy(zbuf, den_sh.at[pl.ds(zbase, half)])
  pltpu.sync_copy(zbuf, den_sh.at[pl.ds(zbase + half, half)])
  pltpu.sync_copy(zbuf, msg_sh.at[pl.ds(zbase, half)])
  pltpu.sync_copy(zbuf, msg_sh.at[pl.ds(zbase + half, half)])
  pltpu.sync_copy(atth, attb)
  plsc.subcore_barrier()

  att = attb[...]
  perms = [jnp.arange(LANES, dtype=jnp.int32) ^ x for x in xors]

  def blk(b, carry):
    base = wid * EPW + b * EB
    pltpu.sync_copy(srcp.at[pl.ds(base, EB)], sidx)
    pltpu.sync_copy(dstp.at[pl.ds(base, EB)], didx)
    pltpu.async_copy(xl.at[sidx], xlb, sem1).wait()
    pltpu.async_copy(xr.at[didx], xrb, sem2).wait()

    def edge(i, ecarry):
      vl = xlb[i, :]
      sv = vl + xrb[i, :]
      lk = jnp.where(sv > 0, sv, sv * jnp.float32(0.2))
      p = lk * att
      for perm in perms:
        p = p + _shuf(p, perm)
      ex = jnp.exp(p)
      exb[i, :] = ex
      msgb[i, :] = ex * vl
      return ecarry

    lax.fori_loop(0, EB, edge, 0)
    pltpu.sync_copy(exb, den_sh.at[didx], add=True)
    pltpu.sync_copy(msgb, msg_sh.at[didx], add=True)
    return carry

  lax.fori_loop(0, NBLK, blk, 0)
  plsc.subcore_barrier()

  pltpu.sync_copy(den_sh.at[pl.ds(zbase, ROWS_PER_TILE)],
                  den_out.at[c, pl.ds(zbase, ROWS_PER_TILE)])
  pltpu.sync_copy(msg_sh.at[pl.ds(zbase, ROWS_PER_TILE)],
                  msg_out.at[c, pl.ds(zbase, ROWS_PER_TILE)])


def _make_edge_kernel(xors):
  mesh = plsc.VectorSubcoreMesh(core_axis_name="c", subcore_axis_name="s",
                                num_cores=NC, num_subcores=NS)
  return pl.kernel(
      functools.partial(_edge_kernel_body, xors),
      out_type=[
          jax.ShapeDtypeStruct((NC, ACC_ROWS, HID), jnp.float32),
          jax.ShapeDtypeStruct((NC, ACC_ROWS, HID), jnp.float32),
      ],
      mesh=mesh,
      scratch_types=[
          pltpu.VMEM((LANES,), jnp.float32),
          pltpu.VMEM((EB,), jnp.int32),
          pltpu.VMEM((EB,), jnp.int32),
          pltpu.VMEM((EB, HID), jnp.float32),
          pltpu.VMEM((EB, HID), jnp.float32),
          pltpu.VMEM((EB, HID), jnp.float32),
          pltpu.VMEM((EB, HID), jnp.float32),
          pltpu.VMEM((ROWS_PER_TILE // 2, HID), jnp.float32),
          pltpu.VMEM_SHARED((ACC_ROWS, HID), jnp.float32),
          pltpu.VMEM_SHARED((ACC_ROWS, HID), jnp.float32),
          pltpu.SemaphoreType.DMA,
          pltpu.SemaphoreType.DMA,
      ],
      compiler_params=pltpu.CompilerParams(use_tc_tiling_on_sc=False),
      name="gat_edge_pass",
  )


_edge_l1 = _make_edge_kernel((1, 2))
_edge_l2 = _make_edge_kernel((1, 2, 4, 8))


def _proj1_body(x_ref, w_ref, b_ref, ol_ref, or_ref):
  acc = jnp.dot(x_ref[...], w_ref[...],
                preferred_element_type=jnp.float32) + b_ref[...]
  ol_ref[...] = acc[:, :HID]
  or_ref[...] = acc[:, HID:]


_PROJ_ROWS = 512
_NROWS1 = 10240


def _proj1(xpad, wcat, bcat):
  return pl.pallas_call(
      _proj1_body,
      grid=(_NROWS1 // _PROJ_ROWS,),
      in_specs=[
          pl.BlockSpec((_PROJ_ROWS, D_IN), lambda i: (i, 0)),
          pl.BlockSpec((D_IN, 2 * HID), lambda i: (0, 0)),
          pl.BlockSpec((1, 2 * HID), lambda i: (0, 0)),
      ],
      out_specs=[
          pl.BlockSpec((_PROJ_ROWS, HID), lambda i: (i, 0)),
          pl.BlockSpec((_PROJ_ROWS, HID), lambda i: (i, 0)),
      ],
      out_shape=[
          jax.ShapeDtypeStruct((_NROWS1, HID), jnp.float32),
          jax.ShapeDtypeStruct((_NROWS1, HID), jnp.float32),
      ],
  )(xpad, wcat, bcat)


_FUSE_ROWS = 1000


def _fuse_body(den_ref, msg_ref, b1_ref, w_ref, b2_ref, ol_ref, or_ref):
  den = den_ref[0] + den_ref[1]
  msg = msg_ref[0] + msg_ref[1]
  h = jnp.maximum(msg / (den + 1e-16) + b1_ref[...], 0.0)
  acc = jnp.dot(h, w_ref[...], preferred_element_type=jnp.float32) + b2_ref[...]
  ol_ref[...] = acc[:, :HID]
  or_ref[...] = acc[:, HID:]


def _fuse(den, msg, bias1, wcat2, bcat2):
  return pl.pallas_call(
      _fuse_body,
      grid=(N // _FUSE_ROWS,),
      in_specs=[
          pl.BlockSpec((NC, _FUSE_ROWS, HID), lambda i: (0, i, 0)),
          pl.BlockSpec((NC, _FUSE_ROWS, HID), lambda i: (0, i, 0)),
          pl.BlockSpec((1, HID), lambda i: (0, 0)),
          pl.BlockSpec((HID, 2 * HID), lambda i: (0, 0)),
          pl.BlockSpec((1, 2 * HID), lambda i: (0, 0)),
      ],
      out_specs=[
          pl.BlockSpec((_FUSE_ROWS, HID), lambda i: (i, 0)),
          pl.BlockSpec((_FUSE_ROWS, HID), lambda i: (i, 0)),
      ],
      out_shape=[
          jax.ShapeDtypeStruct((N, HID), jnp.float32),
          jax.ShapeDtypeStruct((N, HID), jnp.float32),
      ],
  )(den, msg, bias1, wcat2, bcat2)


def _final_body(den_ref, msg_ref, b_ref, o_ref):
  den = den_ref[0] + den_ref[1]
  msg = msg_ref[0] + msg_ref[1]
  o_ref[...] = msg / (den + 1e-16) + b_ref[...]


def _final(den, msg, bias2):
  return pl.pallas_call(
      _final_body,
      grid=(N // _FUSE_ROWS,),
      in_specs=[
          pl.BlockSpec((NC, _FUSE_ROWS, HID), lambda i: (0, i, 0)),
          pl.BlockSpec((NC, _FUSE_ROWS, HID), lambda i: (0, i, 0)),
          pl.BlockSpec((1, HID), lambda i: (0, 0)),
      ],
      out_specs=pl.BlockSpec((_FUSE_ROWS, HID), lambda i: (i, 0)),
      out_shape=jax.ShapeDtypeStruct((N, HID), jnp.float32),
  )(den, msg, bias2)


@jax.jit
def _impl(x, edge_index, Wl1, bl1, Wr1, br1, att1, bias1,
          Wl2, bl2, Wr2, br2, att2, bias2):
  loop = jnp.arange(N, dtype=edge_index.dtype)
  pad = jnp.full((ETOT_PAD - ETOT,), N, dtype=edge_index.dtype)
  srcp = jnp.concatenate([edge_index[0], loop, pad])
  dstp = jnp.concatenate([edge_index[1], loop, pad])

  xpad = jnp.pad(x, ((0, _NROWS1 - N), (0, 0)))
  w1 = jnp.concatenate([Wl1, Wr1], axis=1)
  b1 = jnp.concatenate([bl1, br1]).reshape(1, 2 * HID)
  xl1, xr1 = _proj1(xpad, w1, b1)
  xl1 = xl1[:NPAD]
  xr1 = xr1[:NPAD]

  att1v = att1.reshape(HID)
  den1, msg1 = _edge_l1(srcp, dstp, xl1, xr1, att1v)

  w2 = jnp.concatenate([Wl2, Wr2], axis=1)
  b2 = jnp.concatenate([bl2, br2]).reshape(1, 2 * HID)
  xl2, xr2 = _fuse(den1, msg1, bias1.reshape(1, HID), w2, b2)
  xl2 = jnp.pad(xl2, ((0, NPAD - N), (0, 0)))
  xr2 = jnp.pad(xr2, ((0, NPAD - N), (0, 0)))

  att2v = att2.reshape(HID)
  den2, msg2 = _edge_l2(srcp, dstp, xl2, xr2, att2v)

  return _final(den2, msg2, bias2.reshape(1, HID))


def kernel(x, edge_index, Wl1, bl1, Wr1, br1, att1, bias1,
           Wl2, bl2, Wr2, br2, att2, bias2):
  return _impl(x, edge_index, Wl1, bl1, Wr1, br1, att1, bias1,
               Wl2, bl2, Wr2, br2, att2, bias2)


# speedup vs baseline: 164.2225x; 2.2260x over previous
import functools

import jax
import jax.numpy as jnp
from jax import lax
from jax.experimental import pallas as pl
from jax.experimental.pallas import tpu as pltpu
from jax.experimental.pallas import tpu_sc as plsc

N = 10000
E = 320000
D_IN = 128
HID = 16

NC = 2
NS = 16
NW = NC * NS
LANES = 16

EB = 128
ETOT = E + N
NBLK = 82
EPW = NBLK * EB
ETOT_PAD = EPW * NW
NPAD = 10016
ACC_ROWS = 10240
ROWS_PER_TILE = ACC_ROWS // NS

_GDN = lax.GatherDimensionNumbers(
    offset_dims=(), collapsed_slice_dims=(0,), start_index_map=(0,))


def _shuf(v, perm):
  return lax.gather(v, perm.reshape(LANES, 1), dimension_numbers=_GDN,
                    slice_sizes=(1,),
                    mode=lax.GatherScatterMode.PROMISE_IN_BOUNDS)


def _edge_kernel_body(xors, srcp, dstp, xl, xr, atth, den_out, msg_out,
                      attb, sidx, didx, xlb0, xrb0, xlb1, xrb1,
                      exb0, msgb0, exb1, msgb1, zbuf,
                      den_sh, msg_sh, semg0, semg1, sems0, sems1):
  c = lax.axis_index("c")
  s = lax.axis_index("s")
  wid = s * NC + c

  xlbs = (xlb0, xlb1)
  xrbs = (xrb0, xrb1)
  exbs = (exb0, exb1)
  msgbs = (msgb0, msgb1)
  semgs = (semg0, semg1)
  semss = (sems0, sems1)

  zero = jnp.zeros((LANES,), jnp.float32)

  def zb(i, carry):
    zbuf[i, :] = zero
    return carry

  lax.fori_loop(0, ROWS_PER_TILE // 2, zb, 0)
  zbase = s * ROWS_PER_TILE
  half = ROWS_PER_TILE // 2
  pltpu.sync_copy(zbuf, den_sh.at[pl.ds(zbase, half)])
  pltpu.sync_copy(zbuf, den_sh.at[pl.ds(zbase + half, half)])
  pltpu.sync_copy(zbuf, msg_sh.at[pl.ds(zbase, half)])
  pltpu.sync_copy(zbuf, msg_sh.at[pl.ds(zbase + half, half)])
  pltpu.sync_copy(atth, attb)
  pltpu.sync_copy(srcp.at[wid], sidx)
  pltpu.sync_copy(dstp.at[wid], didx)
  plsc.subcore_barrier()

  att = attb[...]
  perms = [jnp.arange(LANES, dtype=jnp.int32) ^ x for x in xors]

  def gather_start(g, slot):
    pltpu.make_async_copy(xl.at[sidx.at[g]], xlbs[slot], semgs[slot]).start()
    pltpu.make_async_copy(xr.at[didx.at[g]], xrbs[slot], semgs[slot]).start()

  def gather_wait(slot):
    pltpu.make_async_copy(xl.at[sidx.at[0]], xlbs[slot], semgs[slot]).wait()
    pltpu.make_async_copy(xr.at[didx.at[0]], xrbs[slot], semgs[slot]).wait()

  def scatter_start(g, slot):
    pltpu.make_async_copy(exbs[slot], den_sh.at[didx.at[g]],
                          semss[slot]).start(add=True)
    pltpu.make_async_copy(msgbs[slot], msg_sh.at[didx.at[g]],
                          semss[slot]).start(add=True)

  def scatter_wait(slot):
    pltpu.make_async_copy(exbs[slot], den_sh.at[didx.at[0]],
                          semss[slot]).wait()
    pltpu.make_async_copy(msgbs[slot], msg_sh.at[didx.at[0]],
                          semss[slot]).wait()

  def compute(slot):
    xlb, xrb, exb, msgb = xlbs[slot], xrbs[slot], exbs[slot], msgbs[slot]

    def edge(i, ecarry):
      vl = xlb[i, :]
      sv = vl + xrb[i, :]
      lk = jnp.where(sv > 0, sv, sv * jnp.float32(0.2))
      p = lk * att
      for perm in perms:
        p = p + _shuf(p, perm)
      ex = jnp.exp(p)
      exb[i, :] = ex
      msgb[i, :] = ex * vl
      return ecarry

    lax.fori_loop(0, EB, edge, 0)

  gather_start(0, 0)

  def macro(m, carry):
    g0 = 2 * m
    gather_start(g0 + 1, 1)
    gather_wait(0)

    @pl.when(m > 0)
    def _():
      scatter_wait(0)

    compute(0)
    scatter_start(g0, 0)

    @pl.when(g0 + 2 < NBLK)
    def _():
      gather_start(g0 + 2, 0)

    gather_wait(1)

    @pl.when(m > 0)
    def _():
      scatter_wait(1)

    compute(1)
    scatter_start(g0 + 1, 1)
    return carry

  lax.fori_loop(0, NBLK // 2, macro, 0)
  scatter_wait(0)
  scatter_wait(1)
  plsc.subcore_barrier()

  pltpu.sync_copy(den_sh.at[pl.ds(zbase, ROWS_PER_TILE)],
                  den_out.at[c, pl.ds(zbase, ROWS_PER_TILE)])
  pltpu.sync_copy(msg_sh.at[pl.ds(zbase, ROWS_PER_TILE)],
                  msg_out.at[c, pl.ds(zbase, ROWS_PER_TILE)])


def _make_edge_kernel(xors):
  mesh = plsc.VectorSubcoreMesh(core_axis_name="c", subcore_axis_name="s",
                                num_cores=NC, num_subcores=NS)
  return pl.kernel(
      functools.partial(_edge_kernel_body, xors),
      out_type=[
          jax.ShapeDtypeStruct((NC, ACC_ROWS, HID), jnp.float32),
          jax.ShapeDtypeStruct((NC, ACC_ROWS, HID), jnp.float32),
      ],
      mesh=mesh,
      scratch_types=[
          pltpu.VMEM((LANES,), jnp.float32),
          pltpu.VMEM((NBLK, EB), jnp.int32),
          pltpu.VMEM((NBLK, EB), jnp.int32),
          pltpu.VMEM((EB, HID), jnp.float32),
          pltpu.VMEM((EB, HID), jnp.float32),
          pltpu.VMEM((EB, HID), jnp.float32),
          pltpu.VMEM((EB, HID), jnp.float32),
          pltpu.VMEM((EB, HID), jnp.float32),
          pltpu.VMEM((EB, HID), jnp.float32),
          pltpu.VMEM((EB, HID), jnp.float32),
          pltpu.VMEM((EB, HID), jnp.float32),
          pltpu.VMEM((ROWS_PER_TILE // 2, HID), jnp.float32),
          pltpu.VMEM_SHARED((ACC_ROWS, HID), jnp.float32),
          pltpu.VMEM_SHARED((ACC_ROWS, HID), jnp.float32),
          pltpu.SemaphoreType.DMA,
          pltpu.SemaphoreType.DMA,
          pltpu.SemaphoreType.DMA,
          pltpu.SemaphoreType.DMA,
      ],
      compiler_params=pltpu.CompilerParams(use_tc_tiling_on_sc=False),
      name="gat_edge_pass",
  )


_edge_l1 = _make_edge_kernel((1, 2))
_edge_l2 = _make_edge_kernel((1, 2, 4, 8))


def _proj1_body(x_ref, w_ref, b_ref, ol_ref, or_ref):
  acc = jnp.dot(x_ref[...], w_ref[...],
                preferred_element_type=jnp.float32) + b_ref[...]
  ol_ref[...] = acc[:, :HID]
  or_ref[...] = acc[:, HID:]


_PROJ_ROWS = 512
_NROWS1 = 10240


def _proj1(xpad, wcat, bcat):
  return pl.pallas_call(
      _proj1_body,
      grid=(_NROWS1 // _PROJ_ROWS,),
      in_specs=[
          pl.BlockSpec((_PROJ_ROWS, D_IN), lambda i: (i, 0)),
          pl.BlockSpec((D_IN, 2 * HID), lambda i: (0, 0)),
          pl.BlockSpec((1, 2 * HID), lambda i: (0, 0)),
      ],
      out_specs=[
          pl.BlockSpec((_PROJ_ROWS, HID), lambda i: (i, 0)),
          pl.BlockSpec((_PROJ_ROWS, HID), lambda i: (i, 0)),
      ],
      out_shape=[
          jax.ShapeDtypeStruct((_NROWS1, HID), jnp.float32),
          jax.ShapeDtypeStruct((_NROWS1, HID), jnp.float32),
      ],
  )(xpad, wcat, bcat)


_FUSE_ROWS = 1000


def _fuse_body(den_ref, msg_ref, b1_ref, w_ref, b2_ref, ol_ref, or_ref):
  den = den_ref[0] + den_ref[1]
  msg = msg_ref[0] + msg_ref[1]
  h = jnp.maximum(msg / (den + 1e-16) + b1_ref[...], 0.0)
  acc = jnp.dot(h, w_ref[...], preferred_element_type=jnp.float32) + b2_ref[...]
  ol_ref[...] = acc[:, :HID]
  or_ref[...] = acc[:, HID:]


def _fuse(den, msg, bias1, wcat2, bcat2):
  return pl.pallas_call(
      _fuse_body,
      grid=(N // _FUSE_ROWS,),
      in_specs=[
          pl.BlockSpec((NC, _FUSE_ROWS, HID), lambda i: (0, i, 0)),
          pl.BlockSpec((NC, _FUSE_ROWS, HID), lambda i: (0, i, 0)),
          pl.BlockSpec((1, HID), lambda i: (0, 0)),
          pl.BlockSpec((HID, 2 * HID), lambda i: (0, 0)),
          pl.BlockSpec((1, 2 * HID), lambda i: (0, 0)),
      ],
      out_specs=[
          pl.BlockSpec((_FUSE_ROWS, HID), lambda i: (i, 0)),
          pl.BlockSpec((_FUSE_ROWS, HID), lambda i: (i, 0)),
      ],
      out_shape=[
          jax.ShapeDtypeStruct((N, HID), jnp.float32),
          jax.ShapeDtypeStruct((N, HID), jnp.float32),
      ],
  )(den, msg, bias1, wcat2, bcat2)


def _final_body(den_ref, msg_ref, b_ref, o_ref):
  den = den_ref[0] + den_ref[1]
  msg = msg_ref[0] + msg_ref[1]
  o_ref[...] = msg / (den + 1e-16) + b_ref[...]


def _final(den, msg, bias2):
  return pl.pallas_call(
      _final_body,
      grid=(N // _FUSE_ROWS,),
      in_specs=[
          pl.BlockSpec((NC, _FUSE_ROWS, HID), lambda i: (0, i, 0)),
          pl.BlockSpec((NC, _FUSE_ROWS, HID), lambda i: (0, i, 0)),
          pl.BlockSpec((1, HID), lambda i: (0, 0)),
      ],
      out_specs=pl.BlockSpec((_FUSE_ROWS, HID), lambda i: (i, 0)),
      out_shape=jax.ShapeDtypeStruct((N, HID), jnp.float32),
  )(den, msg, bias2)


@jax.jit
def _impl(x, edge_index, Wl1, bl1, Wr1, br1, att1, bias1,
          Wl2, bl2, Wr2, br2, att2, bias2):
  loop = jnp.arange(N, dtype=edge_index.dtype)
  pad = jnp.full((ETOT_PAD - ETOT,), N, dtype=edge_index.dtype)
  srcp = jnp.concatenate([edge_index[0], loop, pad]).reshape(NW, NBLK, EB)
  dstp = jnp.concatenate([edge_index[1], loop, pad]).reshape(NW, NBLK, EB)

  xpad = jnp.pad(x, ((0, _NROWS1 - N), (0, 0)))
  w1 = jnp.concatenate([Wl1, Wr1], axis=1)
  b1 = jnp.concatenate([bl1, br1]).reshape(1, 2 * HID)
  xl1, xr1 = _proj1(xpad, w1, b1)
  xl1 = xl1[:NPAD]
  xr1 = xr1[:NPAD]

  att1v = att1.reshape(HID)
  den1, msg1 = _edge_l1(srcp, dstp, xl1, xr1, att1v)

  w2 = jnp.concatenate([Wl2, Wr2], axis=1)
  b2 = jnp.concatenate([bl2, br2]).reshape(1, 2 * HID)
  xl2, xr2 = _fuse(den1, msg1, bias1.reshape(1, HID), w2, b2)
  xl2 = jnp.pad(xl2, ((0, NPAD - N), (0, 0)))
  xr2 = jnp.pad(xr2, ((0, NPAD - N), (0, 0)))

  att2v = att2.reshape(HID)
  den2, msg2 = _edge_l2(srcp, dstp, xl2, xr2, att2v)

  return _final(den2, msg2, bias2.reshape(1, HID))


def kernel(x, edge_index, Wl1, bl1, Wr1, br1, att1, bias1,
           Wl2, bl2, Wr2, br2, att2, bias2):
  return _impl(x, edge_index, Wl1, bl1, Wr1, br1, att1, bias1,
               Wl2, bl2, Wr2, br2, att2, bias2)
